# baseline (speedup 1.0000x reference)
import functools

import jax
import jax.numpy as jnp
from jax import lax
from jax.experimental import pallas as pl
from jax.experimental.pallas import tpu as pltpu
from jax.experimental.pallas import tpu_sc as plsc

_NUM_CORES = 2
_NUM_SUBCORES = 16
_NW = _NUM_CORES * _NUM_SUBCORES
_CHUNK = 128
_BG = 16


def _sc_gather(uidx, midx, uridx, mridx, userEmb4, movieEmb4, ub16, mb16):
    _, nch, _ = uidx.shape
    B = _NW * nch * _CHUNK
    EMB4 = userEmb4.shape[1]
    bpw = B // _NW

    mesh = plsc.VectorSubcoreMesh(core_axis_name="c", subcore_axis_name="s")

    @functools.partial(
        pl.kernel,
        mesh=mesh,
        compiler_params=pltpu.CompilerParams(use_tc_tiling_on_sc=False),
        out_type=(
            jax.ShapeDtypeStruct((B, EMB4), jnp.float32),
            jax.ShapeDtypeStruct((B, EMB4), jnp.float32),
            jax.ShapeDtypeStruct((B, _BG), jnp.float32),
            jax.ShapeDtypeStruct((B, _BG), jnp.float32),
        ),
        scratch_types=[
            pltpu.VMEM((nch, _CHUNK), jnp.int32),
            pltpu.VMEM((nch, _CHUNK), jnp.int32),
            pltpu.VMEM((nch, _CHUNK), jnp.int32),
            pltpu.VMEM((nch, _CHUNK), jnp.int32),
            pltpu.VMEM((2, _CHUNK, EMB4), jnp.float32),
            pltpu.VMEM((2, _CHUNK, EMB4), jnp.float32),
            pltpu.VMEM((bpw, _BG), jnp.float32),
            pltpu.VMEM((bpw, _BG), jnp.float32),
            pltpu.SemaphoreType.DMA,
            pltpu.SemaphoreType.DMA,
        ],
    )
    def k(uemb_hbm, memb_hbm, ub_hbm, mb_hbm,
          uidx_hbm, midx_hbm, uridx_hbm, mridx_hbm,
          ou, om, oub, omb,
          uidx_v, midx_v, uridx_v, mridx_v, urows2, mrows2, ubv, mbv,
          sem_g, sem_o):
        wid = lax.axis_index("s") * _NUM_CORES + lax.axis_index("c")
        base = wid * bpw
        pltpu.sync_copy(uidx_hbm.at[wid], uidx_v)
        pltpu.sync_copy(midx_hbm.at[wid], midx_v)
        pltpu.sync_copy(uridx_hbm.at[wid], uridx_v)
        pltpu.sync_copy(mridx_hbm.at[wid], mridx_v)
        out_pend = [None, None]
        for j in range(nch):
            b = j & 1
            if out_pend[b] is not None:
                for c in out_pend[b]:
                    c.wait()
            sl = pl.ds(j * _CHUNK, _CHUNK)
            gathers = [
                pltpu.async_copy(uemb_hbm.at[uidx_v.at[j]], urows2.at[b], sem_g),
                pltpu.async_copy(memb_hbm.at[midx_v.at[j]], mrows2.at[b], sem_g),
                pltpu.async_copy(ub_hbm.at[uridx_v.at[j]], ubv.at[sl], sem_g),
                pltpu.async_copy(mb_hbm.at[mridx_v.at[j]], mbv.at[sl], sem_g),
            ]
            for c in gathers:
                c.wait()
            osl = pl.ds(base + j * _CHUNK, _CHUNK)
            out_pend[b] = [
                pltpu.async_copy(urows2.at[b], ou.at[osl], sem_o),
                pltpu.async_copy(mrows2.at[b], om.at[osl], sem_o),
            ]
        for pend in out_pend:
            if pend is not None:
                for c in pend:
                    c.wait()
        osl = pl.ds(base, bpw)
        pltpu.sync_copy(ubv, oub.at[osl])
        pltpu.sync_copy(mbv, omb.at[osl])

    return k(userEmb4, movieEmb4, ub16, mb16, uidx, midx, uridx, mridx)


def _extract32(rows, sel):
    out = jnp.where(sel == 0, rows[:, 0:32], 0.0)
    out = out + jnp.where(sel == 1, rows[:, 32:64], 0.0)
    out = out + jnp.where(sel == 2, rows[:, 64:96], 0.0)
    out = out + jnp.where(sel == 3, rows[:, 96:128], 0.0)
    return out


def _tc_body(const_ref, u_ref, m_ref, goh_ref, ubuf_ref, mbuf_ref,
             ulane_ref, mlane_ref, usel_ref, msel_ref,
             gwt_ref, gb_ref, w1u_ref, w1m_ref, w1g_ref, b1_ref, w2_ref,
             o_ref):
    u = _extract32(u_ref[...], usel_ref[...])
    m = _extract32(m_ref[...], msel_ref[...])
    g = jnp.dot(goh_ref[...], gwt_ref[...], preferred_element_type=jnp.float32)
    g = jnp.maximum(g + gb_ref[...], 0.0)
    h = jnp.dot(u, w1u_ref[...], preferred_element_type=jnp.float32)
    h = h + jnp.dot(m, w1m_ref[...], preferred_element_type=jnp.float32)
    h = h + jnp.dot(g, w1g_ref[...], preferred_element_type=jnp.float32)
    h = jnp.maximum(h + b1_ref[...], 0.0)
    mlp = jnp.sum(h * w2_ref[...], axis=1)
    dot = jnp.sum(u * m, axis=1)
    blk = u.shape[0]
    lanes = lax.broadcasted_iota(jnp.int32, (blk, _BG), 1)
    ub = jnp.sum(jnp.where(lanes == ulane_ref[...], ubuf_ref[...], 0.0), axis=1)
    mb = jnp.sum(jnp.where(lanes == mlane_ref[...], mbuf_ref[...], 0.0), axis=1)
    o_ref[...] = dot + mlp + ub + mb + const_ref[0]


def _tc_dense(urows, mrows, genreOH, ubuf, mbuf, ulane, mlane, usel, msel,
              gW, gb, w1, b1, w2, const):
    B = urows.shape[0]
    EMB = (w1.shape[1] - gW.shape[0]) // 2
    NG = genreOH.shape[1]
    GE = gW.shape[0]
    HL = w1.shape[0]
    BLK = 2048
    grid = (B // BLK,)

    gWt = gW.T
    w1t = w1.T
    w1u = w1t[:EMB]
    w1m = w1t[EMB:2 * EMB]
    w1g = w1t[2 * EMB:]
    w2row = w2[0]

    full = lambda shape: pl.BlockSpec(shape, lambda i: (0,) * len(shape))
    return pl.pallas_call(
        _tc_body,
        grid=grid,
        in_specs=[
            pl.BlockSpec(memory_space=pltpu.SMEM),
            pl.BlockSpec((BLK, 128), lambda i: (i, 0)),
            pl.BlockSpec((BLK, 128), lambda i: (i, 0)),
            pl.BlockSpec((BLK, NG), lambda i: (i, 0)),
            pl.BlockSpec((BLK, _BG), lambda i: (i, 0)),
            pl.BlockSpec((BLK, _BG), lambda i: (i, 0)),
            pl.BlockSpec((BLK, 1), lambda i: (i, 0)),
            pl.BlockSpec((BLK, 1), lambda i: (i, 0)),
            pl.BlockSpec((BLK, 1), lambda i: (i, 0)),
            pl.BlockSpec((BLK, 1), lambda i: (i, 0)),
            full((NG, GE)),
            full((GE,)),
            full((EMB, HL)),
            full((EMB, HL)),
            full((GE, HL)),
            full((HL,)),
            full((HL,)),
        ],
        out_specs=pl.BlockSpec((BLK,), lambda i: (i,)),
        out_shape=jax.ShapeDtypeStruct((B,), jnp.float32),
    )(const, urows, mrows, genreOH, ubuf, mbuf, ulane, mlane, usel, msel,
      gWt, gb, w1u, w1m, w1g, b1, w2row)


def kernel(userOH, moveOH, genreOH, userEmb, movieEmb, userBiasT, movieBiasT,
           bias, gW, gb, w1, b1, w2, b2):
    B = userOH.shape[0]
    nch = B // (_NW * _CHUNK)
    uoh = userOH.astype(jnp.int32)
    moh = moveOH.astype(jnp.int32)
    uidx = (uoh >> 2).reshape(_NW, nch, _CHUNK)
    midx = (moh >> 2).reshape(_NW, nch, _CHUNK)
    uridx = (uoh >> 4).reshape(_NW, nch, _CHUNK)
    mridx = (moh >> 4).reshape(_NW, nch, _CHUNK)
    userEmb4 = userEmb.reshape(userEmb.shape[0] // 4, 128)
    movieEmb4 = movieEmb.reshape(movieEmb.shape[0] // 4, 128)
    ub16 = userBiasT.reshape(userBiasT.shape[0] // _BG, _BG)
    mb16 = movieBiasT.reshape(movieBiasT.shape[0] // _BG, _BG)

    urows, mrows, ubuf, mbuf = _sc_gather(uidx, midx, uridx, mridx,
                                          userEmb4, movieEmb4, ub16, mb16)
    const = (bias + b2).reshape(1)
    col = lambda a: a.reshape(B, 1)
    return _tc_dense(urows, mrows, genreOH, ubuf, mbuf,
                     col(uoh & 15), col(moh & 15),
                     col(uoh & 3), col(moh & 3), gW, gb, w1, b1, w2, const)


# speedup vs baseline: 1.1079x; 1.1079x over previous
import functools

import jax
import jax.numpy as jnp
from jax import lax
from jax.experimental import pallas as pl
from jax.experimental.pallas import tpu as pltpu
from jax.experimental.pallas import tpu_sc as plsc

_NUM_CORES = 2
_NUM_SUBCORES = 16
_NW = _NUM_CORES * _NUM_SUBCORES
_CHUNK = 128


def _sc_gather(uidx, midx, userEmb, movieEmb, ubflat, mbflat):
    _, nch, _ = uidx.shape
    B = _NW * nch * _CHUNK
    EMB = userEmb.shape[1]
    bpw = B // _NW

    mesh = plsc.VectorSubcoreMesh(core_axis_name="c", subcore_axis_name="s")

    @functools.partial(
        pl.kernel,
        mesh=mesh,
        compiler_params=pltpu.CompilerParams(use_tc_tiling_on_sc=False),
        out_type=(
            jax.ShapeDtypeStruct((B, EMB), jnp.float32),
            jax.ShapeDtypeStruct((B, EMB), jnp.float32),
            jax.ShapeDtypeStruct((B,), jnp.float32),
            jax.ShapeDtypeStruct((B,), jnp.float32),
        ),
        scratch_types=[
            pltpu.VMEM((nch, _CHUNK), jnp.int32),
            pltpu.VMEM((nch, _CHUNK), jnp.int32),
            pltpu.VMEM((bpw, EMB), jnp.float32),
            pltpu.VMEM((bpw, EMB), jnp.float32),
            pltpu.VMEM((bpw,), jnp.float32),
            pltpu.VMEM((bpw,), jnp.float32),
            pltpu.SemaphoreType.DMA,
        ],
    )
    def k(uemb_hbm, memb_hbm, ub_hbm, mb_hbm, uidx_hbm, midx_hbm,
          ou, om, oub, omb,
          uidx_v, midx_v, urows, mrows, ub_v, mb_v, sem):
        wid = lax.axis_index("s") * _NUM_CORES + lax.axis_index("c")
        base = wid * bpw
        pltpu.sync_copy(uidx_hbm.at[wid], uidx_v)
        pltpu.sync_copy(midx_hbm.at[wid], midx_v)
        copies = []
        for j in range(nch):
            sl = pl.ds(j * _CHUNK, _CHUNK)
            copies.append(pltpu.async_copy(uemb_hbm.at[uidx_v.at[j]], urows.at[sl], sem))
            copies.append(pltpu.async_copy(memb_hbm.at[midx_v.at[j]], mrows.at[sl], sem))
            copies.append(pltpu.async_copy(ub_hbm.at[uidx_v.at[j]], ub_v.at[sl], sem))
            copies.append(pltpu.async_copy(mb_hbm.at[midx_v.at[j]], mb_v.at[sl], sem))
        for c in copies:
            c.wait()
        osl = pl.ds(base, bpw)
        pltpu.sync_copy(urows, ou.at[osl])
        pltpu.sync_copy(mrows, om.at[osl])
        pltpu.sync_copy(ub_v, oub.at[osl])
        pltpu.sync_copy(mb_v, omb.at[osl])

    return k(userEmb, movieEmb, ubflat, mbflat, uidx, midx)


def _tc_body(const_ref, u_ref, m_ref, goh_ref, ub_ref, mb_ref,
             gwt_ref, gb_ref, w1u_ref, w1m_ref, w1g_ref, b1_ref, w2_ref,
             o_ref):
    u = u_ref[...]
    m = m_ref[...]
    g = jnp.dot(goh_ref[...], gwt_ref[...], preferred_element_type=jnp.float32)
    g = jnp.maximum(g + gb_ref[...], 0.0)
    h = jnp.dot(u, w1u_ref[...], preferred_element_type=jnp.float32)
    h = h + jnp.dot(m, w1m_ref[...], preferred_element_type=jnp.float32)
    h = h + jnp.dot(g, w1g_ref[...], preferred_element_type=jnp.float32)
    h = jnp.maximum(h + b1_ref[...], 0.0)
    mlp = jnp.sum(h * w2_ref[...], axis=1)
    dot = jnp.sum(u * m, axis=1)
    o_ref[...] = dot + mlp + ub_ref[...] + mb_ref[...] + const_ref[0]


def _tc_dense(u, m, genreOH, ub, mb, gW, gb, w1, b1, w2, const):
    B, EMB = u.shape
    NG = genreOH.shape[1]
    GE = gW.shape[0]
    HL = w1.shape[0]
    BLK = 2048
    grid = (B // BLK,)

    gWt = gW.T
    w1t = w1.T
    w1u = w1t[:EMB]
    w1m = w1t[EMB:2 * EMB]
    w1g = w1t[2 * EMB:]
    w2row = w2[0]

    full = lambda shape: pl.BlockSpec(shape, lambda i: (0,) * len(shape))
    return pl.pallas_call(
        _tc_body,
        grid=grid,
        in_specs=[
            pl.BlockSpec(memory_space=pltpu.SMEM),
            pl.BlockSpec((BLK, EMB), lambda i: (i, 0)),
            pl.BlockSpec((BLK, EMB), lambda i: (i, 0)),
            pl.BlockSpec((BLK, NG), lambda i: (i, 0)),
            pl.BlockSpec((BLK,), lambda i: (i,)),
            pl.BlockSpec((BLK,), lambda i: (i,)),
            full((NG, GE)),
            full((GE,)),
            full((EMB, HL)),
            full((EMB, HL)),
            full((GE, HL)),
            full((HL,)),
            full((HL,)),
        ],
        out_specs=pl.BlockSpec((BLK,), lambda i: (i,)),
        out_shape=jax.ShapeDtypeStruct((B,), jnp.float32),
    )(const, u, m, genreOH, ub, mb, gWt, gb, w1u, w1m, w1g, b1, w2row)


def kernel(userOH, moveOH, genreOH, userEmb, movieEmb, userBiasT, movieBiasT,
           bias, gW, gb, w1, b1, w2, b2):
    B = userOH.shape[0]
    nch = B // (_NW * _CHUNK)
    uidx = userOH.astype(jnp.int32).reshape(_NW, nch, _CHUNK)
    midx = moveOH.astype(jnp.int32).reshape(_NW, nch, _CHUNK)
    ubflat = userBiasT.reshape(-1)
    mbflat = movieBiasT.reshape(-1)

    u, m, ub, mb = _sc_gather(uidx, midx, userEmb, movieEmb, ubflat, mbflat)
    const = (bias + b2).reshape(1)
    return _tc_dense(u, m, genreOH, ub, mb, gW, gb, w1, b1, w2, const)
